# baseline (speedup 1.0000x reference)
import functools

import jax
import jax.numpy as jnp
from jax import lax
from jax.experimental import pallas as pl
from jax.experimental.pallas import tpu as pltpu
from jax.experimental.pallas import tpu_sc as plsc

N_NODES = 10000
D_FEAT = 256
HALF = 128
EMBED_DIM = 128
N_REL = 3
N_EDGES = 160000
BATCH = 50000

NC = 2
NS = 16
NW = NC * NS

CH = 64
E_PAD = 163840
E_CHUNKS = E_PAD // NS // CH
G_CH = 128
DUMP = N_NODES
NP = 10240
ROWS_PER_TILE = NP // NS
SLABS = 2
SLAB = NP // SLABS
SLAB_T = SLAB + 16
SROWS = SLAB // NS
ZR = 8

B_PAD = 53248
G_CHUNKS = B_PAD // NW // G_CH


def _sc_segment_kernel(f_both, src_both, dst_hbm, summed, counts,
                       accum, cnts, src_v, dst_v, dloc_v, rows_v, ones_v,
                       zcnt, sem):
  cid = lax.axis_index("c")
  sid = lax.axis_index("s")

  def _init_ones(i, _):
    ones_v[i] = jnp.ones((16,), jnp.float32)
    return ()
  lax.fori_loop(0, CH, _init_ones, ())

  def _init_zcnt(i, _):
    zcnt[i] = jnp.zeros((16,), jnp.float32)
    return ()
  lax.fori_loop(0, CH, _init_zcnt, ())

  for r in range(N_REL):
    for slab in range(SLABS):
      lo = slab * SLAB
      r0 = sid * SROWS

      def _fill_zero(k, _):
        rows_v[k // 8, pl.ds((k % 8) * 16, 16)] = jnp.zeros((16,),
                                                            jnp.float32)
        return ()
      lax.fori_loop(0, CH * 8, _fill_zero, ())

      def _zero(i, _):
        pltpu.sync_copy(rows_v, accum.at[pl.ds(r0 + i * CH, CH)])
        pltpu.sync_copy(zcnt, cnts.at[pl.ds(r0 + i * CH, CH)])
        return ()
      lax.fori_loop(0, SROWS // CH, _zero, ())

      plsc.subcore_barrier()

      sbase = cid * (N_REL * E_PAD) + r * E_PAD + sid * (E_PAD // NS)
      dbase = r * E_PAD + sid * (E_PAD // NS)

      def _chunk(k, _):
        pltpu.sync_copy(src_both.at[pl.ds(sbase + k * CH, CH)], src_v)
        pltpu.sync_copy(dst_hbm.at[pl.ds(dbase + k * CH, CH)], dst_v)
        for t in range(CH // 16):
          d = dst_v[pl.ds(t * 16, 16)] - lo
          ok = (d >= 0) & (d < SLAB)
          dloc_v[pl.ds(t * 16, 16)] = jnp.where(ok, d, SLAB)
        pltpu.async_copy(f_both.at[src_v], rows_v, sem).wait()
        pltpu.sync_copy(rows_v, accum.at[dloc_v], add=True)
        pltpu.sync_copy(ones_v, cnts.at[dloc_v], add=True)
        return ()
      lax.fori_loop(0, E_CHUNKS, _chunk, ())

      plsc.subcore_barrier()

      def _wout(i, _):
        pltpu.sync_copy(accum.at[pl.ds(r0 + i * CH, CH)], rows_v)
        pltpu.sync_copy(
            rows_v,
            summed.at[pl.ds((r * NC + cid) * NP + lo + r0 + i * CH, CH)])
        pltpu.sync_copy(cnts.at[pl.ds(r0 + i * CH, CH)], zcnt)
        pltpu.sync_copy(
            zcnt,
            counts.at[pl.ds((cid * N_REL + r) * NP + lo + r0 + i * CH,
                            CH)])
        return ()
      lax.fori_loop(0, SROWS // CH, _wout, ())

      def _refill_zcnt(i, _):
        zcnt[i] = jnp.zeros((16,), jnp.float32)
        return ()
      lax.fori_loop(0, CH, _refill_zcnt, ())

      plsc.subcore_barrier()


def _sc_segment(f_both, src_both, dst_flat):
  mesh = plsc.VectorSubcoreMesh(core_axis_name="c", subcore_axis_name="s")
  return pl.kernel(
      _sc_segment_kernel,
      out_type=(
          jax.ShapeDtypeStruct((N_REL * NC * NP, HALF), jnp.float32),
          jax.ShapeDtypeStruct((NC * N_REL * NP, 16), jnp.float32),
      ),
      mesh=mesh,
      scratch_types=[
          pltpu.VMEM_SHARED((SLAB_T, HALF), jnp.float32),
          pltpu.VMEM_SHARED((SLAB_T, 16), jnp.float32),
          pltpu.VMEM((CH,), jnp.int32),
          pltpu.VMEM((CH,), jnp.int32),
          pltpu.VMEM((CH,), jnp.int32),
          pltpu.VMEM((CH, HALF), jnp.float32),
          pltpu.VMEM((CH, 16), jnp.float32),
          pltpu.VMEM((CH, 16), jnp.float32),
          pltpu.SemaphoreType.DMA,
      ],
  )(f_both, src_both, dst_flat)


def _sc_gather_kernel(u_hbm, idx_hbm, out_hbm, idx_v, rows_v, sem):
  wid = lax.axis_index("s") * NC + lax.axis_index("c")
  base = wid * (B_PAD // NW)

  def _chunk(k, _):
    b = base + k * G_CH
    pltpu.sync_copy(idx_hbm.at[pl.ds(b, G_CH)], idx_v)
    pltpu.async_copy(u_hbm.at[idx_v], rows_v, sem).wait()
    pltpu.sync_copy(rows_v, out_hbm.at[pl.ds(b, G_CH)])
    return ()
  lax.fori_loop(0, G_CHUNKS, _chunk, ())


def _sc_gather(u, nodes_pad):
  mesh = plsc.VectorSubcoreMesh(core_axis_name="c", subcore_axis_name="s")
  return pl.kernel(
      _sc_gather_kernel,
      out_type=jax.ShapeDtypeStruct((B_PAD, EMBED_DIM), jnp.float32),
      mesh=mesh,
      scratch_types=[
          pltpu.VMEM((G_CH,), jnp.int32),
          pltpu.VMEM((G_CH, EMBED_DIM), jnp.float32),
          pltpu.SemaphoreType.DMA,
      ],
  )(u, nodes_pad)


BLK = 1024


def _tc_dense_kernel(f_ref, s_ref, c_ref, w1_ref, b1_ref, w2_ref, b2_ref,
                     o_ref):
  f = f_ref[...]
  t = jnp.dot(f, w1_ref[0:D_FEAT, :], preferred_element_type=jnp.float32)
  for r in range(N_REL):
    cnt = jnp.maximum(c_ref[r, :, 0:1], 1.0)
    lo = s_ref[r, 0] / cnt
    hi = s_ref[r, 1] / cnt
    base = D_FEAT * (r + 1)
    t += jnp.dot(lo, w1_ref[base:base + HALF, :],
                 preferred_element_type=jnp.float32)
    t += jnp.dot(hi, w1_ref[base + HALF:base + D_FEAT, :],
                 preferred_element_type=jnp.float32)
  h = jnp.tanh(t + b1_ref[...])
  o_ref[...] = jnp.dot(h, w2_ref[...],
                       preferred_element_type=jnp.float32) + b2_ref[...]


def _tc_dense(features, summed, counts, W1, b1, W2, b2):
  grid = (NP // BLK,)
  return pl.pallas_call(
      _tc_dense_kernel,
      grid=grid,
      in_specs=[
          pl.BlockSpec((BLK, D_FEAT), lambda i: (i, 0)),
          pl.BlockSpec((N_REL, NC, BLK, HALF), lambda i: (0, 0, i, 0)),
          pl.BlockSpec((N_REL, BLK, 16), lambda i: (0, i, 0)),
          pl.BlockSpec(((N_REL + 1) * D_FEAT, D_FEAT), lambda i: (0, 0)),
          pl.BlockSpec((1, D_FEAT), lambda i: (0, 0)),
          pl.BlockSpec((D_FEAT, EMBED_DIM), lambda i: (0, 0)),
          pl.BlockSpec((1, EMBED_DIM), lambda i: (0, 0)),
      ],
      out_specs=pl.BlockSpec((BLK, EMBED_DIM), lambda i: (i, 0)),
      out_shape=jax.ShapeDtypeStruct((NP, EMBED_DIM), jnp.float32),
  )(features, summed, counts, W1, b1, W2, b2)


def kernel(nodes, features, edge_index_0, edge_index_1, edge_index_2,
           W1, b1, W2, b2):
  f_both = jnp.concatenate([features[:, :HALF], features[:, HALF:]], axis=0)

  srcs = []
  dsts = []
  pad = E_PAD - N_EDGES
  for e in (edge_index_0, edge_index_1, edge_index_2):
    srcs.append(jnp.concatenate(
        [e[0].astype(jnp.int32), jnp.zeros((pad,), jnp.int32)]))
    dsts.append(jnp.concatenate(
        [e[1].astype(jnp.int32), jnp.full((pad,), DUMP, jnp.int32)]))
  src_flat = jnp.concatenate(srcs)
  src_both = jnp.concatenate([src_flat, src_flat + N_NODES])
  dst_flat = jnp.concatenate(dsts)

  summed_flat, counts_flat = _sc_segment(f_both, src_both, dst_flat)
  summed = summed_flat.reshape(N_REL, NC, NP, HALF)
  counts = counts_flat.reshape(NC, N_REL, NP, 16)[0]

  u = _tc_dense(features, summed, counts, W1,
                b1.reshape(1, D_FEAT), W2, b2.reshape(1, EMBED_DIM))

  nodes_pad = jnp.concatenate(
      [nodes.astype(jnp.int32),
       jnp.zeros((B_PAD - BATCH,), jnp.int32)])
  out = _sc_gather(u, nodes_pad)
  return out[:BATCH]
